# Initial kernel scaffold; baseline (speedup 1.0000x reference)
import jax
import jax.numpy as jnp
from jax.experimental import pallas as pl


def kernel(src, seg, token_table, pos_table, seg_table):
    raise NotImplementedError("write your pallas kernel here")


# speedup vs baseline: 3.9361x; 3.9361x over previous
import functools

import jax
import jax.numpy as jnp
from jax import lax
from jax.experimental import pallas as pl
from jax.experimental.pallas import tpu as pltpu
from jax.experimental.pallas import tpu_sc as plsc

D_MODEL = 64
NUM_CORES = 2
NUM_SUBCORES = 16
NUM_WORKERS = NUM_CORES * NUM_SUBCORES
IDX_PER_STREAM = 128
K_STREAMS = 4
CHUNK = IDX_PER_STREAM * K_STREAMS


def _build_kernel(n_tokens):
    steps = n_tokens // (NUM_WORKERS * CHUNK)
    mesh = plsc.VectorSubcoreMesh(core_axis_name="c", subcore_axis_name="s")

    @functools.partial(
        pl.kernel,
        mesh=mesh,
        compiler_params=pltpu.CompilerParams(use_tc_tiling_on_sc=False),
        out_type=jax.ShapeDtypeStruct((n_tokens, D_MODEL), jnp.float32),
        scratch_types=[
            pltpu.VMEM((K_STREAMS, IDX_PER_STREAM), jnp.int32),
            pltpu.VMEM((K_STREAMS, IDX_PER_STREAM), jnp.int32),
            pltpu.VMEM((CHUNK, D_MODEL), jnp.float32),
            pltpu.VMEM((CHUNK, D_MODEL), jnp.float32),
            pltpu.SemaphoreType.DMA,
        ],
    )
    def emb_kernel(src_hbm, ps_hbm, tok_tab_hbm, ps_tab_hbm, out_hbm,
                   idx_v, psidx_v, rows_v, psrows_v, sem):
        wid = lax.axis_index("s") * NUM_CORES + lax.axis_index("c")

        def step(s, carry):
            g = wid * steps + s
            base = g * CHUNK
            pltpu.sync_copy(src_hbm.at[g], idx_v)
            pltpu.sync_copy(ps_hbm.at[g], psidx_v)
            copies = []
            for j in range(K_STREAMS):
                dst = rows_v.at[pl.ds(j * IDX_PER_STREAM, IDX_PER_STREAM)]
                copies.append(pltpu.async_copy(tok_tab_hbm.at[idx_v.at[j]], dst, sem))
            for j in range(K_STREAMS):
                dst = psrows_v.at[pl.ds(j * IDX_PER_STREAM, IDX_PER_STREAM)]
                copies.append(pltpu.async_copy(ps_tab_hbm.at[psidx_v.at[j]], dst, sem))
            for c in copies:
                c.wait()

            def add_row(t, carry2):
                for l in range(D_MODEL // 16):
                    sl = pl.ds(l * 16, 16)
                    rows_v[t, sl] = rows_v[t, sl] + psrows_v[t, sl]
                return carry2

            lax.fori_loop(0, CHUNK, add_row, 0)
            pltpu.sync_copy(rows_v, out_hbm.at[pl.ds(base, CHUNK)])
            return carry

        lax.fori_loop(0, steps, step, 0)

    return emb_kernel


def kernel(src, seg, token_table, pos_table, seg_table):
    batch, seq_len = src.shape
    n_tokens = batch * seq_len
    ps_table = (pos_table[:seq_len, None, :] + seg_table[None, :, :]).reshape(
        2 * seq_len, D_MODEL)
    pos_ids = jnp.arange(seq_len, dtype=jnp.int32)
    ps_idx = (pos_ids[None, :] * 2 + seg.astype(jnp.int32)).reshape(-1)
    src_blocks = src.astype(jnp.int32).reshape(-1, K_STREAMS, IDX_PER_STREAM)
    ps_blocks = ps_idx.reshape(-1, K_STREAMS, IDX_PER_STREAM)
    emb = _build_kernel(n_tokens)
    out = emb(src_blocks, ps_blocks, token_table, ps_table)
    return out.reshape(batch, seq_len, D_MODEL)


# speedup vs baseline: 3.9627x; 1.0067x over previous
import functools

import jax
import jax.numpy as jnp
from jax import lax
from jax.experimental import pallas as pl
from jax.experimental.pallas import tpu as pltpu
from jax.experimental.pallas import tpu_sc as plsc

D_MODEL = 64
NUM_CORES = 2
NUM_SUBCORES = 16
NUM_WORKERS = NUM_CORES * NUM_SUBCORES
IDX_PER_STREAM = 128
K_STREAMS = 2
CHUNK = IDX_PER_STREAM * K_STREAMS


def _build_kernel(n_tokens):
    steps = n_tokens // (NUM_WORKERS * CHUNK)
    assert steps % 2 == 0
    mesh = plsc.VectorSubcoreMesh(core_axis_name="c", subcore_axis_name="s")

    @functools.partial(
        pl.kernel,
        mesh=mesh,
        compiler_params=pltpu.CompilerParams(use_tc_tiling_on_sc=False),
        out_type=jax.ShapeDtypeStruct((n_tokens, D_MODEL), jnp.float32),
        scratch_types=[
            pltpu.VMEM((2, K_STREAMS, IDX_PER_STREAM), jnp.int32),
            pltpu.VMEM((2, K_STREAMS, IDX_PER_STREAM), jnp.int32),
            pltpu.VMEM((2, CHUNK, D_MODEL), jnp.float32),
            pltpu.VMEM((2, CHUNK, D_MODEL), jnp.float32),
            pltpu.SemaphoreType.DMA,
            pltpu.SemaphoreType.DMA,
            pltpu.SemaphoreType.DMA,
            pltpu.SemaphoreType.DMA,
        ],
    )
    def emb_kernel(src_hbm, ps_hbm, tok_tab_hbm, ps_tab_hbm, out_hbm,
                   idx_v, psidx_v, rows_v, psrows_v,
                   gsem0, gsem1, osem0, osem1):
        wid = lax.axis_index("s") * NUM_CORES + lax.axis_index("c")
        gsems = (gsem0, gsem1)
        osems = (osem0, osem1)

        def fire(s, b):
            g = wid * steps + s
            pltpu.sync_copy(src_hbm.at[g], idx_v.at[b])
            pltpu.sync_copy(ps_hbm.at[g], psidx_v.at[b])
            for j in range(K_STREAMS):
                sl = pl.ds(j * IDX_PER_STREAM, IDX_PER_STREAM)
                pltpu.async_copy(tok_tab_hbm.at[idx_v.at[b, j]],
                                 rows_v.at[b, sl], gsems[b])
                pltpu.async_copy(ps_tab_hbm.at[psidx_v.at[b, j]],
                                 psrows_v.at[b, sl], gsems[b])

        def wait_gathers(b):
            for j in range(K_STREAMS):
                sl = pl.ds(j * IDX_PER_STREAM, IDX_PER_STREAM)
                pltpu.make_async_copy(tok_tab_hbm.at[idx_v.at[b, j]],
                                      rows_v.at[b, sl], gsems[b]).wait()
                pltpu.make_async_copy(ps_tab_hbm.at[psidx_v.at[b, j]],
                                      psrows_v.at[b, sl], gsems[b]).wait()

        def wait_out(b):
            pltpu.make_async_copy(rows_v.at[b],
                                  out_hbm.at[pl.ds(0, CHUNK)], osems[b]).wait()

        def add_and_emit(s, b):
            def add_row(t, carry2):
                for l in range(D_MODEL // 16):
                    sl = pl.ds(l * 16, 16)
                    rows_v[b, t, sl] = rows_v[b, t, sl] + psrows_v[b, t, sl]
                return carry2

            lax.fori_loop(0, CHUNK, add_row, 0)
            base = (wid * steps + s) * CHUNK
            pltpu.async_copy(rows_v.at[b], out_hbm.at[pl.ds(base, CHUNK)],
                             osems[b])

        fire(0, 0)

        def body(i, carry):
            h0 = 2 * i
            wait_gathers(0)

            @pl.when(h0 + 1 < steps)
            def _():
                fire(h0 + 1, 1)

            add_and_emit(h0, 0)
            wait_gathers(1)

            @pl.when(h0 + 2 < steps)
            def _():
                wait_out(0)
                fire(h0 + 2, 0)

            add_and_emit(h0 + 1, 1)

            @pl.when(h0 + 3 < steps)
            def _():
                wait_out(1)

            return carry

        lax.fori_loop(0, steps // 2, body, 0)
        wait_out(0)
        wait_out(1)

    return emb_kernel


def kernel(src, seg, token_table, pos_table, seg_table):
    batch, seq_len = src.shape
    n_tokens = batch * seq_len
    ps_table = (pos_table[:seq_len, None, :] + seg_table[None, :, :]).reshape(
        2 * seq_len, D_MODEL)
    pos_ids = jnp.arange(seq_len, dtype=jnp.int32)
    ps_idx = (pos_ids[None, :] * 2 + seg.astype(jnp.int32)).reshape(-1)
    src_blocks = src.astype(jnp.int32).reshape(-1, K_STREAMS, IDX_PER_STREAM)
    ps_blocks = ps_idx.reshape(-1, K_STREAMS, IDX_PER_STREAM)
    emb = _build_kernel(n_tokens)
    out = emb(src_blocks, ps_blocks, token_table, ps_table)
    return out.reshape(batch, seq_len, D_MODEL)


# speedup vs baseline: 4.1949x; 1.0586x over previous
import functools

import jax
import jax.numpy as jnp
from jax import lax
from jax.experimental import pallas as pl
from jax.experimental.pallas import tpu as pltpu
from jax.experimental.pallas import tpu_sc as plsc

D_MODEL = 64
NUM_CORES = 2
NUM_SUBCORES = 16
NUM_WORKERS = NUM_CORES * NUM_SUBCORES
SEQ = 200
IDX_PER_STREAM = 100
K_STREAMS = 4
ROWS_PER_CHUNK = 2
CHUNK = IDX_PER_STREAM * K_STREAMS


def _build_kernel(batch):
    rows_per_worker = batch // NUM_WORKERS
    steps = rows_per_worker // ROWS_PER_CHUNK
    assert steps % 2 == 0
    mesh = plsc.VectorSubcoreMesh(core_axis_name="c", subcore_axis_name="s")

    @functools.partial(
        pl.kernel,
        mesh=mesh,
        compiler_params=pltpu.CompilerParams(use_tc_tiling_on_sc=False),
        out_type=jax.ShapeDtypeStruct((batch, SEQ, D_MODEL), jnp.float32),
        scratch_types=[
            pltpu.VMEM((2, K_STREAMS, IDX_PER_STREAM), jnp.int32),
            pltpu.VMEM((2, K_STREAMS, IDX_PER_STREAM), jnp.int32),
            pltpu.VMEM((2, ROWS_PER_CHUNK, SEQ, D_MODEL), jnp.float32),
            pltpu.VMEM((2, ROWS_PER_CHUNK, SEQ, D_MODEL), jnp.float32),
            pltpu.SemaphoreType.DMA,
            pltpu.SemaphoreType.DMA,
            pltpu.SemaphoreType.DMA,
            pltpu.SemaphoreType.DMA,
        ],
    )
    def emb_kernel(src_hbm, ps_hbm, tok_tab_hbm, ps_tab_hbm, out_hbm,
                   idx_v, psidx_v, rows_v, psrows_v,
                   gsem0, gsem1, osem0, osem1):
        cid = lax.axis_index("c")
        sid = lax.axis_index("s")
        wid = sid * NUM_CORES + cid
        gsems = (gsem0, gsem1)
        osems = (osem0, osem1)

        def fire(s, b):
            g = wid * steps + s
            pltpu.sync_copy(src_hbm.at[g], idx_v.at[b])
            pltpu.sync_copy(ps_hbm.at[g], psidx_v.at[b])
            for j in range(K_STREAMS):
                r, c = j // 2, (j % 2) * IDX_PER_STREAM
                dst = rows_v.at[b, r, pl.ds(c, IDX_PER_STREAM)]
                psdst = psrows_v.at[b, r, pl.ds(c, IDX_PER_STREAM)]
                pltpu.async_copy(tok_tab_hbm.at[idx_v.at[b, j]], dst, gsems[b])
                pltpu.async_copy(ps_tab_hbm.at[psidx_v.at[b, j]], psdst, gsems[b])

        def wait_gathers(b):
            for j in range(K_STREAMS):
                r, c = j // 2, (j % 2) * IDX_PER_STREAM
                dst = rows_v.at[b, r, pl.ds(c, IDX_PER_STREAM)]
                psdst = psrows_v.at[b, r, pl.ds(c, IDX_PER_STREAM)]
                pltpu.make_async_copy(tok_tab_hbm.at[idx_v.at[b, j]], dst,
                                      gsems[b]).wait()
                pltpu.make_async_copy(ps_tab_hbm.at[psidx_v.at[b, j]], psdst,
                                      gsems[b]).wait()

        def wait_out(b):
            pltpu.make_async_copy(rows_v.at[b],
                                  out_hbm.at[pl.ds(0, ROWS_PER_CHUNK)],
                                  osems[b]).wait()

        def add_and_emit(s, b):
            def add_row(t, carry2):
                for r in range(ROWS_PER_CHUNK):
                    for l in range(D_MODEL // 16):
                        sl = pl.ds(l * 16, 16)
                        rows_v[b, r, t, sl] = (rows_v[b, r, t, sl]
                                               + psrows_v[b, r, t, sl])
                return carry2

            lax.fori_loop(0, SEQ, add_row, 0)
            row0 = wid * rows_per_worker + s * ROWS_PER_CHUNK
            pltpu.async_copy(rows_v.at[b],
                             out_hbm.at[pl.ds(row0, ROWS_PER_CHUNK)], osems[b])

        fire(0, 0)

        def body(i, carry):
            h0 = 2 * i
            wait_gathers(0)

            @pl.when(h0 + 1 < steps)
            def _():
                fire(h0 + 1, 1)

            add_and_emit(h0, 0)
            wait_gathers(1)

            @pl.when(h0 + 2 < steps)
            def _():
                wait_out(0)
                fire(h0 + 2, 0)

            add_and_emit(h0 + 1, 1)

            @pl.when(h0 + 3 < steps)
            def _():
                wait_out(1)

            return carry

        lax.fori_loop(0, steps // 2, body, 0)
        wait_out(0)
        wait_out(1)

    return emb_kernel


def kernel(src, seg, token_table, pos_table, seg_table):
    batch, seq_len = src.shape
    ps_table = (pos_table[:seq_len, None, :] + seg_table[None, :, :]).reshape(
        2 * seq_len, D_MODEL)
    pos_ids = jnp.arange(seq_len, dtype=jnp.int32)
    ps_idx = (pos_ids[None, :] * 2 + seg.astype(jnp.int32)).reshape(-1)
    src_blocks = src.astype(jnp.int32).reshape(-1, K_STREAMS, IDX_PER_STREAM)
    ps_blocks = ps_idx.reshape(-1, K_STREAMS, IDX_PER_STREAM)
    emb = _build_kernel(batch)
    return emb(src_blocks, ps_blocks, token_table, ps_table)
